# Initial kernel scaffold; baseline (speedup 1.0000x reference)
import jax
import jax.numpy as jnp
from jax.experimental import pallas as pl


def kernel(x, edge_index, W11, b11, W12, b12, W21, b21, W22, b22, W31, b31, W32, b32, Wa, ba, Wb, bb, Wc, bc, Wr, br, Wcls, bcls):
    raise NotImplementedError("write your pallas kernel here")


# speedup vs baseline: 3.5321x; 3.5321x over previous
import functools

import jax
import jax.numpy as jnp
from jax import lax
from jax.experimental import pallas as pl
from jax.experimental.pallas import tpu as pltpu
from jax.experimental.pallas import tpu_sc as plsc

_CHUNK = 128


def _segsum_body(x_ref, srcp, dstp, idx_s, idx_d, rows, acc, sem,
                 base, n_chunks):
    def body(k, carry):
        off = base + k * _CHUNK
        pltpu.sync_copy(srcp.at[pl.ds(off, _CHUNK)], idx_s)
        pltpu.sync_copy(dstp.at[pl.ds(off, _CHUNK)], idx_d)
        pltpu.async_copy(x_ref.at[idx_s], rows, sem).wait()
        pltpu.sync_copy(rows, acc.at[idx_d], add=True)
        return carry
    lax.fori_loop(0, n_chunks, body, 0)


@functools.lru_cache(maxsize=None)
def _make_segsum_colsplit(n_pad, d_half, t_tile):
    n_chunks = t_tile // _CHUNK
    rows_stripe = n_pad // 16
    mesh = plsc.VectorSubcoreMesh(core_axis_name="c", subcore_axis_name="s")

    @functools.partial(
        pl.kernel,
        mesh=mesh,
        out_type=[jax.ShapeDtypeStruct((n_pad, d_half), jnp.float32)] * 2,
        scratch_types=[
            pltpu.VMEM((_CHUNK,), jnp.int32),
            pltpu.VMEM((_CHUNK,), jnp.int32),
            pltpu.VMEM((_CHUNK, d_half), jnp.float32),
            pltpu.VMEM_SHARED((n_pad, d_half), jnp.float32),
            pltpu.SemaphoreType.DMA,
        ],
    )
    def segsum(xlo, xhi, srcp, dstp, zeros, out_lo, out_hi,
               idx_s, idx_d, rows, acc, sem):
        c = lax.axis_index("c")
        s = lax.axis_index("s")
        stripe = pl.ds(s * rows_stripe, rows_stripe)

        pltpu.sync_copy(zeros.at[stripe], acc.at[stripe])
        plsc.subcore_barrier()

        base = s * t_tile

        @pl.when(c == 0)
        def _():
            _segsum_body(xlo, srcp, dstp, idx_s, idx_d, rows, acc, sem,
                         base, n_chunks)

        @pl.when(c == 1)
        def _():
            _segsum_body(xhi, srcp, dstp, idx_s, idx_d, rows, acc, sem,
                         base, n_chunks)

        plsc.subcore_barrier()

        @pl.when(c == 0)
        def _():
            pltpu.sync_copy(acc.at[stripe], out_lo.at[stripe])

        @pl.when(c == 1)
        def _():
            pltpu.sync_copy(acc.at[stripe], out_hi.at[stripe])

    return segsum


@functools.lru_cache(maxsize=None)
def _make_segsum_edgesplit(n_pad, d, t_tile):
    n_chunks = t_tile // _CHUNK
    rows_stripe = n_pad // 16
    mesh = plsc.VectorSubcoreMesh(core_axis_name="c", subcore_axis_name="s")

    @functools.partial(
        pl.kernel,
        mesh=mesh,
        out_type=[jax.ShapeDtypeStruct((n_pad, d), jnp.float32)] * 2,
        scratch_types=[
            pltpu.VMEM((_CHUNK,), jnp.int32),
            pltpu.VMEM((_CHUNK,), jnp.int32),
            pltpu.VMEM((_CHUNK, d), jnp.float32),
            pltpu.VMEM_SHARED((n_pad, d), jnp.float32),
            pltpu.SemaphoreType.DMA,
        ],
    )
    def segsum(x, srcp, dstp, zeros, out_p0, out_p1,
               idx_s, idx_d, rows, acc, sem):
        c = lax.axis_index("c")
        s = lax.axis_index("s")
        stripe = pl.ds(s * rows_stripe, rows_stripe)

        pltpu.sync_copy(zeros.at[stripe], acc.at[stripe])
        plsc.subcore_barrier()

        base = (c * 16 + s) * t_tile
        _segsum_body(x, srcp, dstp, idx_s, idx_d, rows, acc, sem,
                     base, n_chunks)

        plsc.subcore_barrier()

        @pl.when(c == 0)
        def _():
            pltpu.sync_copy(acc.at[stripe], out_p0.at[stripe])

        @pl.when(c == 1)
        def _():
            pltpu.sync_copy(acc.at[stripe], out_p1.at[stripe])

    return segsum


def _mlp_call(n, dims, build_u, args, w1, b1, w2, b2):
    d_in = w1.shape[0]
    h = w2.shape[1]
    bn = 1000

    def body(*refs):
        arg_refs = refs[:len(args)]
        w1_r, b1_r, w2_r, b2_r, ol_r, oh_r = refs[len(args):]
        u = build_u(*arg_refs)
        t = jnp.maximum(
            jnp.dot(u, w1_r[...], preferred_element_type=jnp.float32)
            + b1_r[...], 0.0)
        out = jnp.maximum(
            jnp.dot(t, w2_r[...], preferred_element_type=jnp.float32)
            + b2_r[...], 0.0)
        ol_r[...] = out[:, :h // 2]
        oh_r[...] = out[:, h // 2:]

    row_spec = lambda d: pl.BlockSpec((bn, d), lambda i: (i, 0))
    full_spec = lambda a, b: pl.BlockSpec((a, b), lambda i: (0, 0))
    return pl.pallas_call(
        body,
        grid=(n // bn,),
        in_specs=[row_spec(d) for d in dims] +
                 [full_spec(d_in, h), full_spec(1, h),
                  full_spec(h, h), full_spec(1, h)],
        out_specs=[pl.BlockSpec((bn, h // 2), lambda i: (i, 0))] * 2,
        out_shape=[jax.ShapeDtypeStruct((n, h // 2), jnp.float32)] * 2,
    )(*args, w1, b1, w2, b2)


def _mlp_partials(x, p0, p1, w1, b1, w2, b2):
    d = x.shape[1]
    build = lambda x_r, p0_r, p1_r: x_r[...] + p0_r[...] + p1_r[...]
    return _mlp_call(x.shape[0], [d, d, d], build, (x, p0, p1),
                     w1, b1, w2, b2)


def _mlp_halves(xl, xh, al, ah, w1, b1, w2, b2):
    dh = xl.shape[1]
    build = lambda xl_r, xh_r, al_r, ah_r: jnp.concatenate(
        [xl_r[...] + al_r[...], xh_r[...] + ah_r[...]], axis=1)
    return _mlp_call(xl.shape[0], [dh, dh, dh, dh], build, (xl, xh, al, ah),
                     w1, b1, w2, b2)


def _attn_head(xl, xh, wa, ba, wb, bb, wc_t, bc, wr, br, wcls, bcls):
    n, dh = xl.shape
    h = 2 * dh
    c_out = wcls.shape[1]
    bn = 1000
    nb = n // bn

    def body(xl_r, xh_r, wa_r, ba_r, wb_r, bb_r, wc_r, bc_r,
             wr_r, br_r, wcls_r, bcls_r,
             logits_r, haz_r, surv_r, m_ref, l_ref, acc_ref):
        i = pl.program_id(0)
        x3 = jnp.concatenate([xl_r[...], xh_r[...]], axis=1)
        a = jnp.tanh(
            jnp.dot(x3, wa_r[...], preferred_element_type=jnp.float32)
            + ba_r[...])
        g = jax.nn.sigmoid(
            jnp.dot(x3, wb_r[...], preferred_element_type=jnp.float32)
            + bb_r[...])
        s = jnp.sum((a * g) * wc_r[...], axis=1, keepdims=True) + bc_r[...]

        @pl.when(i == 0)
        def _():
            m_ref[0, 0] = -1e30
            l_ref[0, 0] = 0.0
            acc_ref[...] = jnp.zeros_like(acc_ref)

        m_old = m_ref[0, 0]
        m_new = jnp.maximum(m_old, jnp.max(s))
        corr = jnp.exp(m_old - m_new)
        w = jnp.exp(s - m_new)
        l_new = l_ref[0, 0] * corr + jnp.sum(w)
        acc_new = acc_ref[...] * corr + lax.dot_general(
            w, x3, (((0,), (0,)), ((), ())),
            preferred_element_type=jnp.float32)
        m_ref[0, 0] = m_new
        l_ref[0, 0] = l_new
        acc_ref[...] = acc_new

        @pl.when(i == nb - 1)
        def _():
            hvec = acc_new / l_new
            h2 = jnp.maximum(
                jnp.dot(hvec, wr_r[...], preferred_element_type=jnp.float32)
                + br_r[...], 0.0)
            lg = jnp.dot(h2, wcls_r[...],
                         preferred_element_type=jnp.float32) + bcls_r[...]
            hz = jax.nn.sigmoid(lg)
            p = 1.0 - hz
            s0 = p[:, 0:1]
            s1 = s0 * p[:, 1:2]
            s2 = s1 * p[:, 2:3]
            s3 = s2 * p[:, 3:4]
            logits_r[...] = lg
            haz_r[...] = hz
            surv_r[...] = jnp.concatenate([s0, s1, s2, s3], axis=1)

    row_spec = lambda d: pl.BlockSpec((bn, d), lambda i: (i, 0))
    full_spec = lambda a, b: pl.BlockSpec((a, b), lambda i: (0, 0))
    return pl.pallas_call(
        body,
        grid=(nb,),
        in_specs=[row_spec(dh), row_spec(dh),
                  full_spec(h, h), full_spec(1, h),
                  full_spec(h, h), full_spec(1, h),
                  full_spec(1, h), full_spec(1, 1),
                  full_spec(h, h), full_spec(1, h),
                  full_spec(h, c_out), full_spec(1, c_out)],
        out_specs=[full_spec(1, c_out)] * 3,
        out_shape=[jax.ShapeDtypeStruct((1, c_out), jnp.float32)] * 3,
        scratch_shapes=[
            pltpu.SMEM((1, 1), jnp.float32),
            pltpu.SMEM((1, 1), jnp.float32),
            pltpu.VMEM((1, h), jnp.float32),
        ],
    )(xl, xh, wa, ba, wb, bb, wc_t, bc, wr, br, wcls, bcls)


def kernel(x, edge_index, W11, b11, W12, b12, W21, b21, W22, b22,
           W31, b31, W32, b32, Wa, ba, Wb, bb, Wc, bc, Wr, br, Wcls, bcls):
    n, d_in = x.shape
    h = W12.shape[1]
    e = edge_index.shape[1]

    e_pad = -(-e // (32 * _CHUNK)) * (32 * _CHUNK)
    src = edge_index[0]
    dst = edge_index[1]
    if e_pad > e:
        pad = e_pad - e
        src = jnp.concatenate([src, jnp.zeros((pad,), jnp.int32)])
        dst = jnp.concatenate([dst, jnp.full((pad,), n, jnp.int32)])
    n_pad = -(-(n + 1) // 128) * 128

    z_in = jnp.zeros((n_pad, d_in), jnp.float32)
    z_h = jnp.zeros((n_pad, h // 2), jnp.float32)

    seg1 = _make_segsum_edgesplit(n_pad, d_in, e_pad // 32)
    seg_h = _make_segsum_colsplit(n_pad, h // 2, e_pad // 16)

    b11r, b12r = b11.reshape(1, -1), b12.reshape(1, -1)
    b21r, b22r = b21.reshape(1, -1), b22.reshape(1, -1)
    b31r, b32r = b31.reshape(1, -1), b32.reshape(1, -1)

    p0, p1 = seg1(x, src, dst, z_in)
    h1l, h1h = _mlp_partials(x, p0, p1, W11, b11r, W12, b12r)
    a2l, a2h = seg_h(h1l, h1h, src, dst, z_h)
    h2l, h2h = _mlp_halves(h1l, h1h, a2l, a2h, W21, b21r, W22, b22r)
    a3l, a3h = seg_h(h2l, h2h, src, dst, z_h)
    h3l, h3h = _mlp_halves(h2l, h2h, a3l, a3h, W31, b31r, W32, b32r)

    return _attn_head(h3l, h3h,
                      Wa, ba.reshape(1, -1), Wb, bb.reshape(1, -1),
                      Wc.reshape(1, -1), bc.reshape(1, 1),
                      Wr, br.reshape(1, -1), Wcls, bcls.reshape(1, -1))


# speedup vs baseline: 3.6369x; 1.0297x over previous
import functools

import jax
import jax.numpy as jnp
from jax import lax
from jax.experimental import pallas as pl
from jax.experimental.pallas import tpu as pltpu
from jax.experimental.pallas import tpu_sc as plsc

_CHUNK = 128


_IDXG = 16


def _segsum_body(x_ref, epk, idxg, rows, acc, sems, chunk_base, n_chunks):
    n_groups = n_chunks // _IDXG

    def gather(idx_slice, r, sem):
        pltpu.make_async_copy(x_ref.at[idx_slice], r, sem).start()

    def drain(r, sem):
        pltpu.make_async_copy(x_ref.at[idxg[0].at[0, 0]], r, sem).wait()

    def scat(r, idx_slice):
        pltpu.sync_copy(r, acc.at[idx_slice], add=True)

    pltpu.sync_copy(epk.at[pl.ds(chunk_base, _IDXG)], idxg[0])
    gather(idxg[0].at[0, 0], rows[0], sems[0])
    gather(idxg[0].at[1, 0], rows[1], sems[1])

    for g in range(n_groups):
        cur = idxg[g % 2]
        nxt = idxg[(g + 1) % 2]
        last = g + 1 == n_groups
        if not last:
            pltpu.sync_copy(
                epk.at[pl.ds(chunk_base + (g + 1) * _IDXG, _IDXG)], nxt)

        def pair(j, carry, cur=cur):
            c0 = 2 * j
            drain(rows[0], sems[0])
            scat(rows[0], cur.at[c0, 1])
            gather(cur.at[c0 + 2, 0], rows[0], sems[0])
            drain(rows[1], sems[1])
            scat(rows[1], cur.at[c0 + 1, 1])
            gather(cur.at[c0 + 3, 0], rows[1], sems[1])
            return carry

        lax.fori_loop(0, _IDXG // 2 - 1, pair, 0)

        drain(rows[0], sems[0])
        scat(rows[0], cur.at[_IDXG - 2, 1])
        if not last:
            gather(nxt.at[0, 0], rows[0], sems[0])
        drain(rows[1], sems[1])
        scat(rows[1], cur.at[_IDXG - 1, 1])
        if not last:
            gather(nxt.at[1, 0], rows[1], sems[1])


@functools.lru_cache(maxsize=None)
def _make_segsum_colsplit(n_pad, d_half, t_tile):
    n_chunks = t_tile // _CHUNK
    rows_stripe = n_pad // 16
    mesh = plsc.VectorSubcoreMesh(core_axis_name="c", subcore_axis_name="s")

    @functools.partial(
        pl.kernel,
        mesh=mesh,
        out_type=[jax.ShapeDtypeStruct((n_pad, d_half), jnp.float32)] * 2,
        scratch_types=[
            pltpu.VMEM((_IDXG, 2, _CHUNK), jnp.int32),
            pltpu.VMEM((_IDXG, 2, _CHUNK), jnp.int32),
            pltpu.VMEM((_CHUNK, d_half), jnp.float32),
            pltpu.VMEM((_CHUNK, d_half), jnp.float32),
            pltpu.VMEM_SHARED((n_pad, d_half), jnp.float32),
            pltpu.SemaphoreType.DMA,
            pltpu.SemaphoreType.DMA,
        ],
    )
    def segsum(xlo, xhi, epk, zeros, out_lo, out_hi,
               idxg_a, idxg_b, rows_a, rows_b, acc, sem_a, sem_b):
        c = lax.axis_index("c")
        s = lax.axis_index("s")
        stripe = pl.ds(s * rows_stripe, rows_stripe)

        pltpu.sync_copy(zeros.at[stripe], acc.at[stripe])
        plsc.subcore_barrier()

        @pl.when(c == 0)
        def _():
            _segsum_body(xlo, epk, (idxg_a, idxg_b), (rows_a, rows_b), acc,
                         (sem_a, sem_b), s * n_chunks, n_chunks)

        @pl.when(c == 1)
        def _():
            _segsum_body(xhi, epk, (idxg_a, idxg_b), (rows_a, rows_b), acc,
                         (sem_a, sem_b), s * n_chunks, n_chunks)

        plsc.subcore_barrier()

        @pl.when(c == 0)
        def _():
            pltpu.sync_copy(acc.at[stripe], out_lo.at[stripe])

        @pl.when(c == 1)
        def _():
            pltpu.sync_copy(acc.at[stripe], out_hi.at[stripe])

    return segsum


@functools.lru_cache(maxsize=None)
def _make_segsum_edgesplit(n_pad, d, t_tile):
    n_chunks = t_tile // _CHUNK
    rows_stripe = n_pad // 16
    mesh = plsc.VectorSubcoreMesh(core_axis_name="c", subcore_axis_name="s")

    @functools.partial(
        pl.kernel,
        mesh=mesh,
        out_type=[jax.ShapeDtypeStruct((n_pad, d), jnp.float32)] * 2,
        scratch_types=[
            pltpu.VMEM((_IDXG, 2, _CHUNK), jnp.int32),
            pltpu.VMEM((_IDXG, 2, _CHUNK), jnp.int32),
            pltpu.VMEM((_CHUNK, d), jnp.float32),
            pltpu.VMEM((_CHUNK, d), jnp.float32),
            pltpu.VMEM_SHARED((n_pad, d), jnp.float32),
            pltpu.SemaphoreType.DMA,
            pltpu.SemaphoreType.DMA,
        ],
    )
    def segsum(x, epk, zeros, out_p0, out_p1,
               idxg_a, idxg_b, rows_a, rows_b, acc, sem_a, sem_b):
        c = lax.axis_index("c")
        s = lax.axis_index("s")
        stripe = pl.ds(s * rows_stripe, rows_stripe)

        pltpu.sync_copy(zeros.at[stripe], acc.at[stripe])
        plsc.subcore_barrier()

        _segsum_body(x, epk, (idxg_a, idxg_b), (rows_a, rows_b), acc,
                     (sem_a, sem_b), (c * 16 + s) * n_chunks, n_chunks)

        plsc.subcore_barrier()

        @pl.when(c == 0)
        def _():
            pltpu.sync_copy(acc.at[stripe], out_p0.at[stripe])

        @pl.when(c == 1)
        def _():
            pltpu.sync_copy(acc.at[stripe], out_p1.at[stripe])

    return segsum


def _mlp_call(n, dims, build_u, args, w1, b1, w2, b2):
    d_in = w1.shape[0]
    h = w2.shape[1]
    bn = 1000

    def body(*refs):
        arg_refs = refs[:len(args)]
        w1_r, b1_r, w2_r, b2_r, ol_r, oh_r = refs[len(args):]
        u = build_u(*arg_refs)
        t = jnp.maximum(
            jnp.dot(u, w1_r[...], preferred_element_type=jnp.float32)
            + b1_r[...], 0.0)
        out = jnp.maximum(
            jnp.dot(t, w2_r[...], preferred_element_type=jnp.float32)
            + b2_r[...], 0.0)
        ol_r[...] = out[:, :h // 2]
        oh_r[...] = out[:, h // 2:]

    row_spec = lambda d: pl.BlockSpec((bn, d), lambda i: (i, 0))
    full_spec = lambda a, b: pl.BlockSpec((a, b), lambda i: (0, 0))
    return pl.pallas_call(
        body,
        grid=(n // bn,),
        in_specs=[row_spec(d) for d in dims] +
                 [full_spec(d_in, h), full_spec(1, h),
                  full_spec(h, h), full_spec(1, h)],
        out_specs=[pl.BlockSpec((bn, h // 2), lambda i: (i, 0))] * 2,
        out_shape=[jax.ShapeDtypeStruct((n, h // 2), jnp.float32)] * 2,
    )(*args, w1, b1, w2, b2)


def _mlp_partials(x, p0, p1, w1, b1, w2, b2):
    d = x.shape[1]
    build = lambda x_r, p0_r, p1_r: x_r[...] + p0_r[...] + p1_r[...]
    return _mlp_call(x.shape[0], [d, d, d], build, (x, p0, p1),
                     w1, b1, w2, b2)


def _mlp_halves(xl, xh, al, ah, w1, b1, w2, b2):
    dh = xl.shape[1]
    build = lambda xl_r, xh_r, al_r, ah_r: jnp.concatenate(
        [xl_r[...] + al_r[...], xh_r[...] + ah_r[...]], axis=1)
    return _mlp_call(xl.shape[0], [dh, dh, dh, dh], build, (xl, xh, al, ah),
                     w1, b1, w2, b2)


def _attn_head(xl, xh, wa, ba, wb, bb, wc_t, bc, wr, br, wcls, bcls):
    n, dh = xl.shape
    h = 2 * dh
    c_out = wcls.shape[1]
    bn = 1000
    nb = n // bn

    def body(xl_r, xh_r, wa_r, ba_r, wb_r, bb_r, wc_r, bc_r,
             wr_r, br_r, wcls_r, bcls_r,
             logits_r, haz_r, surv_r, m_ref, l_ref, acc_ref):
        i = pl.program_id(0)
        x3 = jnp.concatenate([xl_r[...], xh_r[...]], axis=1)
        a = jnp.tanh(
            jnp.dot(x3, wa_r[...], preferred_element_type=jnp.float32)
            + ba_r[...])
        g = jax.nn.sigmoid(
            jnp.dot(x3, wb_r[...], preferred_element_type=jnp.float32)
            + bb_r[...])
        s = jnp.sum((a * g) * wc_r[...], axis=1, keepdims=True) + bc_r[...]

        @pl.when(i == 0)
        def _():
            m_ref[0, 0] = -1e30
            l_ref[0, 0] = 0.0
            acc_ref[...] = jnp.zeros_like(acc_ref)

        m_old = m_ref[0, 0]
        m_new = jnp.maximum(m_old, jnp.max(s))
        corr = jnp.exp(m_old - m_new)
        w = jnp.exp(s - m_new)
        l_new = l_ref[0, 0] * corr + jnp.sum(w)
        acc_new = acc_ref[...] * corr + lax.dot_general(
            w, x3, (((0,), (0,)), ((), ())),
            preferred_element_type=jnp.float32)
        m_ref[0, 0] = m_new
        l_ref[0, 0] = l_new
        acc_ref[...] = acc_new

        @pl.when(i == nb - 1)
        def _():
            hvec = acc_new / l_new
            h2 = jnp.maximum(
                jnp.dot(hvec, wr_r[...], preferred_element_type=jnp.float32)
                + br_r[...], 0.0)
            lg = jnp.dot(h2, wcls_r[...],
                         preferred_element_type=jnp.float32) + bcls_r[...]
            hz = jax.nn.sigmoid(lg)
            p = 1.0 - hz
            s0 = p[:, 0:1]
            s1 = s0 * p[:, 1:2]
            s2 = s1 * p[:, 2:3]
            s3 = s2 * p[:, 3:4]
            logits_r[...] = lg
            haz_r[...] = hz
            surv_r[...] = jnp.concatenate([s0, s1, s2, s3], axis=1)

    row_spec = lambda d: pl.BlockSpec((bn, d), lambda i: (i, 0))
    full_spec = lambda a, b: pl.BlockSpec((a, b), lambda i: (0, 0))
    return pl.pallas_call(
        body,
        grid=(nb,),
        in_specs=[row_spec(dh), row_spec(dh),
                  full_spec(h, h), full_spec(1, h),
                  full_spec(h, h), full_spec(1, h),
                  full_spec(1, h), full_spec(1, 1),
                  full_spec(h, h), full_spec(1, h),
                  full_spec(h, c_out), full_spec(1, c_out)],
        out_specs=[full_spec(1, c_out)] * 3,
        out_shape=[jax.ShapeDtypeStruct((1, c_out), jnp.float32)] * 3,
        scratch_shapes=[
            pltpu.SMEM((1, 1), jnp.float32),
            pltpu.SMEM((1, 1), jnp.float32),
            pltpu.VMEM((1, h), jnp.float32),
        ],
    )(xl, xh, wa, ba, wb, bb, wc_t, bc, wr, br, wcls, bcls)


def kernel(x, edge_index, W11, b11, W12, b12, W21, b21, W22, b22,
           W31, b31, W32, b32, Wa, ba, Wb, bb, Wc, bc, Wr, br, Wcls, bcls):
    n, d_in = x.shape
    h = W12.shape[1]
    e = edge_index.shape[1]

    e_pad = -(-e // (64 * _CHUNK)) * (64 * _CHUNK)
    src = edge_index[0]
    dst = edge_index[1]
    if e_pad > e:
        pad = e_pad - e
        src = jnp.concatenate([src, jnp.zeros((pad,), jnp.int32)])
        dst = jnp.concatenate([dst, jnp.full((pad,), n, jnp.int32)])
    epk = jnp.stack([src.reshape(-1, _CHUNK), dst.reshape(-1, _CHUNK)],
                    axis=1)
    n_pad = -(-(n + 1) // 128) * 128

    z_in = jnp.zeros((n_pad, d_in), jnp.float32)
    z_h = jnp.zeros((n_pad, h // 2), jnp.float32)

    seg1 = _make_segsum_edgesplit(n_pad, d_in, e_pad // 32)
    seg_h = _make_segsum_colsplit(n_pad, h // 2, e_pad // 16)

    b11r, b12r = b11.reshape(1, -1), b12.reshape(1, -1)
    b21r, b22r = b21.reshape(1, -1), b22.reshape(1, -1)
    b31r, b32r = b31.reshape(1, -1), b32.reshape(1, -1)

    p0, p1 = seg1(x, epk, z_in)
    h1l, h1h = _mlp_partials(x, p0, p1, W11, b11r, W12, b12r)
    a2l, a2h = seg_h(h1l, h1h, epk, z_h)
    h2l, h2h = _mlp_halves(h1l, h1h, a2l, a2h, W21, b21r, W22, b22r)
    a3l, a3h = seg_h(h2l, h2h, epk, z_h)
    h3l, h3h = _mlp_halves(h2l, h2h, a3l, a3h, W31, b31r, W32, b32r)

    return _attn_head(h3l, h3h,
                      Wa, ba.reshape(1, -1), Wb, bb.reshape(1, -1),
                      Wc.reshape(1, -1), bc.reshape(1, 1),
                      Wr, br.reshape(1, -1), Wcls, bcls.reshape(1, -1))
